# baseline (speedup 1.0000x reference)
import numpy as np
import jax
import jax.numpy as jnp
from jax import lax
from jax.experimental import pallas as pl
from jax.experimental.pallas import tpu as pltpu
from jax.experimental.pallas import tpu_sc as plsc

H = 16
NP = 20000
NL = 2000
T = 8
PPL = 80
PADP = 20480
PADL = 2048
BPAD = 163840
RP = NP // 8
RPS = PADP // 8
RL = NL // 8
RLS = PADL // 8
NC, NS = 2, 16
NW = NC * NS
BPW = BPAD // NW
CH = 128
NCH = BPW // CH
GRIDPTS = np.linspace(-2.0, 2.0, 8).astype(np.float32)

_SDS = jax.ShapeDtypeStruct


def _gather_waves(wid, table_hbm, idx_v, rows_v, out_hbm, gsem, osem):
    pltpu.async_copy(table_hbm.at[idx_v], rows_v, gsem).wait()
    pltpu.async_copy(rows_v, out_hbm.at[pl.ds(wid * BPW, BPW)], osem).wait()


def _sc_mesh():
    return plsc.VectorSubcoreMesh(
        core_axis_name="c", subcore_axis_name="s",
        num_cores=NC, num_subcores=NS)


def _sc_gather_call(table, idx2d):
    v, d = table.shape
    if v * d * 4 <= 160 * 1024:
        return _sc_gather_small_call(table, idx2d)

    def body(table_hbm, idx_hbm, out_hbm, idx_v, rows_v, gsem, osem):
        wid = lax.axis_index("s") * NC + lax.axis_index("c")
        pltpu.sync_copy(idx_hbm.at[pl.ds(wid * BPW, BPW)], idx_v)
        _gather_waves(wid, table_hbm, idx_v, rows_v, out_hbm, gsem, osem)

    fn = pl.kernel(
        body,
        out_type=_SDS((BPAD, d), jnp.float32),
        mesh=_sc_mesh(),
        scratch_types=[
            pltpu.VMEM((BPW,), jnp.int32),
            pltpu.VMEM((BPW, d), jnp.float32),
            pltpu.SemaphoreType.DMA,
            pltpu.SemaphoreType.DMA,
        ],
        compiler_params=pltpu.CompilerParams(use_tc_tiling_on_sc=False, skip_device_barrier=True),
    )
    return fn(table, idx2d.reshape(BPAD))


def _sc_gather_small_call(table, idx2d):
    v, d = table.shape
    ngrp = BPW // 16

    def body(table_hbm, idx_hbm, out_hbm, tab_v, idx_v, rows_v, osem):
        wid = lax.axis_index("s") * NC + lax.axis_index("c")
        pltpu.sync_copy(table_hbm, tab_v)
        pltpu.sync_copy(idx_hbm.at[pl.ds(wid * BPW, BPW)], idx_v)
        lanes = lax.iota(jnp.int32, 16)

        def grp(g, carry):
            rowv = idx_v[pl.ds(g * 16, 16)]
            base = g * (16 * d) + lanes * d
            for j in range(d):
                col = plsc.load_gather(
                    tab_v, [rowv, jnp.full((16,), j, jnp.int32)])
                plsc.store_scatter(rows_v, [base + j], col)
            return carry

        lax.fori_loop(0, ngrp, grp, 0)
        pltpu.async_copy(rows_v,
                         out_hbm.at[pl.ds(wid * BPW * d, BPW * d)],
                         osem).wait()

    fn = pl.kernel(
        body,
        out_type=_SDS((BPAD * d,), jnp.float32),
        mesh=_sc_mesh(),
        scratch_types=[
            pltpu.VMEM((v, d), jnp.float32),
            pltpu.VMEM((BPW,), jnp.int32),
            pltpu.VMEM((BPW * d,), jnp.float32),
            pltpu.SemaphoreType.DMA,
        ],
        compiler_params=pltpu.CompilerParams(
            use_tc_tiling_on_sc=False, skip_device_barrier=True,
            needs_layout_passes=False),
    )
    return fn(table, idx2d.reshape(BPAD)).reshape(BPAD, d)


def _dot(a, b):
    return jnp.dot(a, b, preferred_element_type=jnp.float32)


def _selu(x):
    alpha = 1.6732631921768188
    scale = 1.0507010221481323
    return scale * jnp.where(x > 0, x, alpha * (jnp.exp(x) - 1.0))


def _gru_p(x, h, wz, wr, wh, rz, rr, rh, bz, br, bh):
    z = jax.nn.sigmoid(_dot(x, wz) + _dot(h, rz) + bz)
    r = jax.nn.sigmoid(_dot(x, wr) + _dot(h, rr) + br)
    hh = jnp.tanh(_dot(x, wh) + _dot(r * h, rh) + bh)
    return z * h + (1.0 - z) * hh


def _path_embed_body(x_ref, w1_ref, b1_ref, w2_ref, b2_ref, o_ref):
    h = _selu(_dot(x_ref[...], w1_ref[...]) + b1_ref[...])
    o_ref[...] = _selu(_dot(h, w2_ref[...]) + b2_ref[...])


def _link_embed_body(pgt_ref, cap_ref, mll_ref, deg_ref, msk_ref,
                     w1_ref, b1_ref, w2_ref, b2_ref, o_ref):
    acc = jnp.zeros((RL, 128), jnp.float32)
    for j in range(PPL):
        acc = acc + pgt_ref[pl.ds(j * RLS, RL), :]
    cap = cap_ref[...]
    load = acc / (cap * 1e9)
    nl = load / mll_ref[0, 0]
    lfeat = (cap * msk_ref[0:1, :] + load * msk_ref[1:2, :]
             + nl * msk_ref[2:3, :] + deg_ref[...] * msk_ref[3:4, :])
    h = _selu(_dot(lfeat, w1_ref[...]) + b1_ref[...])
    o_ref[...] = _selu(_dot(h, w2_ref[...]) + b2_ref[...])


def _scan_body(x_ref, h0_ref, wz_ref, wr_ref, wh_ref, rz_ref, rr_ref, rh_ref,
               bz_ref, br_ref, bh_ref, pss_ref, hout_ref):
    wz, wr, wh = wz_ref[...], wr_ref[...], wh_ref[...]
    rz, rr, rh = rz_ref[...], rr_ref[...], rh_ref[...]
    bz, br, bh = bz_ref[...], br_ref[...], bh_ref[...]
    h = h0_ref[...]
    pss_ref[pl.ds(0, RP), :] = h
    for t in range(T):
        x = x_ref[pl.ds(t * RPS, RP), :]
        h = _gru_p(x, h, wz, wr, wh, rz, rr, rh, bz, br, bh)
        pss_ref[pl.ds((t + 1) * RP, RP), :] = h
    hout_ref[...] = h


def _att_body(pg_ref, ls_ref, aw_ref, ab_ref, g_ref,
              wz_ref, wr_ref, wh_ref, rz_ref, rr_ref, rh_ref,
              bz_ref, br_ref, bh_ref, o_ref):
    aw = aw_ref[...]
    ab = ab_ref[...]
    gm = g_ref[...]
    m = jnp.zeros((RL, 128), jnp.float32)
    for j in range(PPL):
        g = pg_ref[pl.ds(j * RLS, RL), :]
        a = _dot(g, aw) + ab
        a = jnp.where(a > 0, a, 0.01 * a)
        e = jnp.exp(a)
        s = e / _dot(e, gm)
        m = m + s * g
    o_ref[...] = _gru_p(m, ls_ref[...],
                        wz_ref[...], wr_ref[...], wh_ref[...],
                        rz_ref[...], rr_ref[...], rh_ref[...],
                        bz_ref[...], br_ref[...], bh_ref[...])


def _readout_body(pss_ref, capg_ref, k1w_ref, k1b_ref, k1s_ref,
                  k2w_ref, k2b_ref, k2s_ref, p_ref, o_ref):
    k1w, k1b = k1w_ref[...], k1b_ref[...]
    k2w, k2b = k2w_ref[...], k2b_ref[...]
    psel = p_ref[...]
    out = jnp.zeros((RP, 8), jnp.float32)
    for t in range(T):
        x = pss_ref[pl.ds((t + 1) * RP, RP), :]
        acc = _dot(jax.nn.silu(x), k1w) + k1b
        for g in range(8):
            phi = jnp.exp(-(x - GRIDPTS[g]) ** 2)
            acc = acc + _dot(phi, k1s_ref[g])
        occ = _dot(jax.nn.silu(acc), k2w) + k2b
        for g in range(8):
            phi = jnp.exp(-(acc - GRIDPTS[g]) ** 2)
            occ = occ + _dot(phi, k2s_ref[g])
        cap8 = _dot(capg_ref[pl.ds(t * RPS, RP), :], psel)
        out = out + occ / cap8
    o_ref[...] = out


def _tc(body, out_shape, *args):
    if isinstance(out_shape, list):
        shapes = [_SDS(s, jnp.float32) for s in out_shape]
    else:
        shapes = _SDS(out_shape, jnp.float32)
    return pl.pallas_call(body, out_shape=shapes)(*args)


def kernel(flow_traffic, flow_packets, flow_packet_size, flow_ibg,
           flow_on_rate, flow_p90PktSize, flow_bitrate_per_burst,
           flow_pkts_per_burst, link_capacity, max_link_load, flow_length,
           link_to_path, path_to_link, devices_to_link, pe_W1, pe_b1, pe_W2,
           pe_b2, le_W1, le_b1, le_W2, le_b2, att_W, att_b, pu_Wk, pu_Rk,
           pu_b, lu_Wk, lu_Rk, lu_b, k1_Wb, k1_Ws, k1_b, k2_Wb, k2_Ws, k2_b):
    f32 = jnp.float32
    i8 = jnp.eye(8, dtype=f32)
    bd = lambda w: jnp.kron(i8, w)
    t16 = lambda b: jnp.tile(b, 8)[None]

    l2p = link_to_path.astype(jnp.int32)
    idx_a = jnp.pad(l2p.T, ((0, 0), (0, PADP - NP))).reshape(NW * NCH, CH)
    f_idx = path_to_link[..., 0].astype(jnp.int32)
    t_idx = path_to_link[..., 1].astype(jnp.int32)
    idx_b = jnp.pad((t_idx * NP + f_idx).T,
                    ((0, 0), (0, PADL - NL))).reshape(NW * NCH, CH)
    idx_c = jnp.pad(f_idx.T, ((0, 0), (0, PADL - NL))).reshape(NW * NCH, CH)

    pfeat = jnp.concatenate(
        [flow_traffic, flow_packets, flow_packet_size, flow_ibg,
         flow_on_rate, flow_p90PktSize, flow_bitrate_per_burst,
         flow_pkts_per_burst, flow_length.astype(f32)[:, None]], axis=1)
    pfeat = jnp.pad(pfeat, ((0, 0), (0, 7))).reshape(RP, 128)
    ndeg = jnp.sum(jnp.ones_like(devices_to_link), axis=1)
    l2d = devices_to_link * 0 + jnp.arange(devices_to_link.shape[0])[:, None]
    gdeg = jnp.take(ndeg, l2d)
    degs = (jnp.reshape(gdeg, (-1,)) / jnp.sum(gdeg)).astype(f32)[:, None]

    ftb = jnp.broadcast_to(flow_traffic, (NP, H))
    capb = jnp.broadcast_to(link_capacity, (NL, H))
    capp = capb.reshape(RL, 128)
    degp = jnp.broadcast_to(degs, (NL, H)).reshape(RL, 128)
    msk = jnp.stack([jnp.tile((jnp.arange(16) == i).astype(f32), 8)
                     for i in range(4)])

    pe_w1 = bd(jnp.pad(pe_W1, ((0, 7), (0, 0))))
    pe_w2 = bd(pe_W2)
    le_w1 = bd(jnp.pad(le_W1, ((0, 12), (0, 0))))
    le_w2 = bd(le_W2)
    pu = [bd(pu_Wk[:, g * H:(g + 1) * H]) for g in range(3)] + \
         [bd(pu_Rk[:, g * H:(g + 1) * H]) for g in range(3)] + \
         [t16(pu_b[g * H:(g + 1) * H]) for g in range(3)]
    lu = [bd(lu_Wk[:, g * H:(g + 1) * H]) for g in range(3)] + \
         [bd(lu_Rk[:, g * H:(g + 1) * H]) for g in range(3)] + \
         [t16(lu_b[g * H:(g + 1) * H]) for g in range(3)]
    aw = bd(att_W)
    ab = t16(att_b)
    gmat = jnp.kron(i8, jnp.ones((16, 16), f32))
    k1w = bd(k1_Wb)
    k1b = jnp.tile(k1_b, 8)[None]
    k1s = jnp.stack([bd(k1_Ws[:, g, :]) for g in range(8)])
    k2w = bd(k2_Wb)
    k2b = jnp.tile(k2_b, 8)[None]
    k2s = jnp.stack([bd(k2_Ws[:, g, :]) for g in range(8)])
    psel = jnp.kron(i8, (jnp.arange(16) == 0).astype(f32)[:, None])

    pgt = _sc_gather_call(ftb, idx_c).reshape(BPAD // 8, 128)
    capg = _sc_gather_call(capb, idx_a).reshape(BPAD // 8, 128)
    ps = _tc(_path_embed_body, (RP, 128),
             pfeat, pe_w1, t16(pe_b1), pe_w2, t16(pe_b2))
    ls = _tc(_link_embed_body, (RL, 128),
             pgt, capp, max_link_load.reshape(1, 1), degp, msk,
             le_w1, t16(le_b1), le_w2, t16(le_b2))

    pss = None
    for _ in range(12):
        xg = _sc_gather_call(ls.reshape(NL, H), idx_a).reshape(BPAD // 8, 128)
        pss, ps = _tc(_scan_body, [(9 * RP, 128), (RP, 128)],
                      xg, ps, *pu)
        pg = _sc_gather_call(pss.reshape(9 * NP, H),
                             idx_b).reshape(BPAD // 8, 128)
        ls = _tc(_att_body, (RL, 128), pg, ls, aw, ab, gmat, *lu)

    out = _tc(_readout_body, (RP, 8),
              pss, capg, k1w, k1b, k1s, k2w, k2b, k2s, psel)
    return out.reshape(NP, 1)


# speedup vs baseline: 1.1874x; 1.1874x over previous
import numpy as np
import jax
import jax.numpy as jnp
from jax import lax
from jax.experimental import pallas as pl
from jax.experimental.pallas import tpu as pltpu
from jax.experimental.pallas import tpu_sc as plsc

H = 16
NP = 20000
NL = 2000
T = 8
PPL = 80
PADP = 20480
PADL = 2048
BPAD = 163840
RP = NP // 8
RPS = PADP // 8
RL = NL // 8
RLS = PADL // 8
NC, NS = 2, 16
NW = NC * NS
BPW = BPAD // NW
CH = 128
NCH = BPW // CH
GRIDPTS = np.linspace(-2.0, 2.0, 8).astype(np.float32)

_SDS = jax.ShapeDtypeStruct


def _gather_waves(wid, table_hbm, idx_v, rows_v, out_hbm, gsem):
    pltpu.async_copy(table_hbm.at[idx_v], rows_v, gsem).wait()
    pltpu.sync_copy(rows_v, out_hbm.at[pl.ds(wid * BPW, BPW)])


def _sc_mesh():
    return plsc.VectorSubcoreMesh(
        core_axis_name="c", subcore_axis_name="s",
        num_cores=NC, num_subcores=NS)


def _sc_gather_call(table, idx2d):
    _, d = table.shape

    def body(table_hbm, idx_hbm, out_hbm, idx_v, rows_v, gsem):
        wid = lax.axis_index("s") * NC + lax.axis_index("c")
        pltpu.sync_copy(idx_hbm.at[pl.ds(wid * BPW, BPW)], idx_v)
        _gather_waves(wid, table_hbm, idx_v, rows_v, out_hbm, gsem)

    fn = pl.kernel(
        body,
        out_type=_SDS((BPAD, d), jnp.float32),
        mesh=_sc_mesh(),
        scratch_types=[
            pltpu.VMEM((BPW,), jnp.int32),
            pltpu.VMEM((BPW, d), jnp.float32),
            pltpu.SemaphoreType.DMA,
        ],
        compiler_params=pltpu.CompilerParams(use_tc_tiling_on_sc=False),
    )
    return fn(table, idx2d.reshape(BPAD))


def _dot(a, b):
    return jnp.dot(a, b, preferred_element_type=jnp.float32)


def _selu(x):
    alpha = 1.6732631921768188
    scale = 1.0507010221481323
    return scale * jnp.where(x > 0, x, alpha * (jnp.exp(x) - 1.0))


def _gru_p(x, h, wz, wr, wh, rz, rr, rh, bz, br, bh):
    z = jax.nn.sigmoid(_dot(x, wz) + _dot(h, rz) + bz)
    r = jax.nn.sigmoid(_dot(x, wr) + _dot(h, rr) + br)
    hh = jnp.tanh(_dot(x, wh) + _dot(r * h, rh) + bh)
    return z * h + (1.0 - z) * hh


def _path_embed_body(x_ref, w1_ref, b1_ref, w2_ref, b2_ref, o_ref):
    h = _selu(_dot(x_ref[...], w1_ref[...]) + b1_ref[...])
    o_ref[...] = _selu(_dot(h, w2_ref[...]) + b2_ref[...])


def _link_embed_body(pgt_ref, cap_ref, mll_ref, deg_ref, msk_ref,
                     w1_ref, b1_ref, w2_ref, b2_ref, o_ref):
    acc = jnp.zeros((RL, 128), jnp.float32)
    for j in range(PPL):
        acc = acc + pgt_ref[pl.ds(j * RLS, RL), :]
    cap = cap_ref[...]
    load = acc / (cap * 1e9)
    nl = load / mll_ref[0, 0]
    lfeat = (cap * msk_ref[0:1, :] + load * msk_ref[1:2, :]
             + nl * msk_ref[2:3, :] + deg_ref[...] * msk_ref[3:4, :])
    h = _selu(_dot(lfeat, w1_ref[...]) + b1_ref[...])
    o_ref[...] = _selu(_dot(h, w2_ref[...]) + b2_ref[...])


def _scan_body(x_ref, h0_ref, wz_ref, wr_ref, wh_ref, rz_ref, rr_ref, rh_ref,
               bz_ref, br_ref, bh_ref, pss_ref, hout_ref):
    wz, wr, wh = wz_ref[...], wr_ref[...], wh_ref[...]
    rz, rr, rh = rz_ref[...], rr_ref[...], rh_ref[...]
    bz, br, bh = bz_ref[...], br_ref[...], bh_ref[...]
    h = h0_ref[...]
    pss_ref[pl.ds(0, RP), :] = h
    for t in range(T):
        x = x_ref[pl.ds(t * RPS, RP), :]
        h = _gru_p(x, h, wz, wr, wh, rz, rr, rh, bz, br, bh)
        pss_ref[pl.ds((t + 1) * RP, RP), :] = h
    hout_ref[...] = h


def _att_body(pg_ref, ls_ref, aw_ref, ab_ref, g_ref,
              wz_ref, wr_ref, wh_ref, rz_ref, rr_ref, rh_ref,
              bz_ref, br_ref, bh_ref, o_ref):
    aw = aw_ref[...]
    ab = ab_ref[...]
    gm = g_ref[...]
    m = jnp.zeros((RL, 128), jnp.float32)
    for j in range(PPL):
        g = pg_ref[pl.ds(j * RLS, RL), :]
        a = _dot(g, aw) + ab
        a = jnp.where(a > 0, a, 0.01 * a)
        e = jnp.exp(a)
        s = e / _dot(e, gm)
        m = m + s * g
    o_ref[...] = _gru_p(m, ls_ref[...],
                        wz_ref[...], wr_ref[...], wh_ref[...],
                        rz_ref[...], rr_ref[...], rh_ref[...],
                        bz_ref[...], br_ref[...], bh_ref[...])


def _readout_body(pss_ref, capg_ref, k1w_ref, k1b_ref, k1s_ref,
                  k2w_ref, k2b_ref, k2s_ref, p_ref, o_ref):
    k1w, k1b = k1w_ref[...], k1b_ref[...]
    k2w, k2b = k2w_ref[...], k2b_ref[...]
    psel = p_ref[...]
    out = jnp.zeros((RP, 8), jnp.float32)
    for t in range(T):
        x = pss_ref[pl.ds((t + 1) * RP, RP), :]
        acc = _dot(jax.nn.silu(x), k1w) + k1b
        for g in range(8):
            phi = jnp.exp(-(x - GRIDPTS[g]) ** 2)
            acc = acc + _dot(phi, k1s_ref[g])
        occ = _dot(jax.nn.silu(acc), k2w) + k2b
        for g in range(8):
            phi = jnp.exp(-(acc - GRIDPTS[g]) ** 2)
            occ = occ + _dot(phi, k2s_ref[g])
        cap8 = _dot(capg_ref[pl.ds(t * RPS, RP), :], psel)
        out = out + occ / cap8
    o_ref[...] = out


def _tc(body, out_shape, *args):
    if isinstance(out_shape, list):
        shapes = [_SDS(s, jnp.float32) for s in out_shape]
    else:
        shapes = _SDS(out_shape, jnp.float32)
    return pl.pallas_call(body, out_shape=shapes)(*args)


def kernel(flow_traffic, flow_packets, flow_packet_size, flow_ibg,
           flow_on_rate, flow_p90PktSize, flow_bitrate_per_burst,
           flow_pkts_per_burst, link_capacity, max_link_load, flow_length,
           link_to_path, path_to_link, devices_to_link, pe_W1, pe_b1, pe_W2,
           pe_b2, le_W1, le_b1, le_W2, le_b2, att_W, att_b, pu_Wk, pu_Rk,
           pu_b, lu_Wk, lu_Rk, lu_b, k1_Wb, k1_Ws, k1_b, k2_Wb, k2_Ws, k2_b):
    f32 = jnp.float32
    i8 = jnp.eye(8, dtype=f32)
    bd = lambda w: jnp.kron(i8, w)
    t16 = lambda b: jnp.tile(b, 8)[None]

    l2p = link_to_path.astype(jnp.int32)
    idx_a = jnp.pad(l2p.T, ((0, 0), (0, PADP - NP))).reshape(NW * NCH, CH)
    f_idx = path_to_link[..., 0].astype(jnp.int32)
    t_idx = path_to_link[..., 1].astype(jnp.int32)
    idx_b = jnp.pad((t_idx * NP + f_idx).T,
                    ((0, 0), (0, PADL - NL))).reshape(NW * NCH, CH)
    idx_c = jnp.pad(f_idx.T, ((0, 0), (0, PADL - NL))).reshape(NW * NCH, CH)

    pfeat = jnp.concatenate(
        [flow_traffic, flow_packets, flow_packet_size, flow_ibg,
         flow_on_rate, flow_p90PktSize, flow_bitrate_per_burst,
         flow_pkts_per_burst, flow_length.astype(f32)[:, None]], axis=1)
    pfeat = jnp.pad(pfeat, ((0, 0), (0, 7))).reshape(RP, 128)
    ndeg = jnp.sum(jnp.ones_like(devices_to_link), axis=1)
    l2d = devices_to_link * 0 + jnp.arange(devices_to_link.shape[0])[:, None]
    gdeg = jnp.take(ndeg, l2d)
    degs = (jnp.reshape(gdeg, (-1,)) / jnp.sum(gdeg)).astype(f32)[:, None]

    ftb = jnp.broadcast_to(flow_traffic, (NP, H))
    capb = jnp.broadcast_to(link_capacity, (NL, H))
    capp = capb.reshape(RL, 128)
    degp = jnp.broadcast_to(degs, (NL, H)).reshape(RL, 128)
    msk = jnp.stack([jnp.tile((jnp.arange(16) == i).astype(f32), 8)
                     for i in range(4)])

    pe_w1 = bd(jnp.pad(pe_W1, ((0, 7), (0, 0))))
    pe_w2 = bd(pe_W2)
    le_w1 = bd(jnp.pad(le_W1, ((0, 12), (0, 0))))
    le_w2 = bd(le_W2)
    pu = [bd(pu_Wk[:, g * H:(g + 1) * H]) for g in range(3)] + \
         [bd(pu_Rk[:, g * H:(g + 1) * H]) for g in range(3)] + \
         [t16(pu_b[g * H:(g + 1) * H]) for g in range(3)]
    lu = [bd(lu_Wk[:, g * H:(g + 1) * H]) for g in range(3)] + \
         [bd(lu_Rk[:, g * H:(g + 1) * H]) for g in range(3)] + \
         [t16(lu_b[g * H:(g + 1) * H]) for g in range(3)]
    aw = bd(att_W)
    ab = t16(att_b)
    gmat = jnp.kron(i8, jnp.ones((16, 16), f32))
    k1w = bd(k1_Wb)
    k1b = jnp.tile(k1_b, 8)[None]
    k1s = jnp.stack([bd(k1_Ws[:, g, :]) for g in range(8)])
    k2w = bd(k2_Wb)
    k2b = jnp.tile(k2_b, 8)[None]
    k2s = jnp.stack([bd(k2_Ws[:, g, :]) for g in range(8)])
    psel = jnp.kron(i8, (jnp.arange(16) == 0).astype(f32)[:, None])

    pgt = _sc_gather_call(ftb, idx_c).reshape(BPAD // 8, 128)
    capg = _sc_gather_call(capb, idx_a).reshape(BPAD // 8, 128)
    ps = _tc(_path_embed_body, (RP, 128),
             pfeat, pe_w1, t16(pe_b1), pe_w2, t16(pe_b2))
    ls = _tc(_link_embed_body, (RL, 128),
             pgt, capp, max_link_load.reshape(1, 1), degp, msk,
             le_w1, t16(le_b1), le_w2, t16(le_b2))

    pss = None
    for _ in range(12):
        xg = _sc_gather_call(ls.reshape(NL, H), idx_a).reshape(BPAD // 8, 128)
        pss, ps = _tc(_scan_body, [(9 * RP, 128), (RP, 128)],
                      xg, ps, *pu)
        pg = _sc_gather_call(pss.reshape(9 * NP, H),
                             idx_b).reshape(BPAD // 8, 128)
        ls = _tc(_att_body, (RL, 128), pg, ls, aw, ab, gmat, *lu)

    out = _tc(_readout_body, (RP, 8),
              pss, capg, k1w, k1b, k1s, k2w, k2b, k2s, psel)
    return out.reshape(NP, 1)


# speedup vs baseline: 1.1883x; 1.0008x over previous
import numpy as np
import jax
import jax.numpy as jnp
from jax import lax
from jax.experimental import pallas as pl
from jax.experimental.pallas import tpu as pltpu
from jax.experimental.pallas import tpu_sc as plsc

H = 16
NP = 20000
NL = 2000
T = 8
PPL = 80
PADP = 20480
PADL = 2048
BPAD = 163840
RP = NP // 8
RPS = PADP // 8
RL = NL // 8
RLS = PADL // 8
NC, NS = 2, 16
NW = NC * NS
BPW = BPAD // NW
CH = 128
NCH = BPW // CH
GRIDPTS = np.linspace(-2.0, 2.0, 8).astype(np.float32)

_SDS = jax.ShapeDtypeStruct


def _gather_waves(wid, table_hbm, idx_v, rows_v, out_hbm, gsem):
    pltpu.async_copy(table_hbm.at[idx_v], rows_v, gsem).wait()
    pltpu.sync_copy(rows_v, out_hbm.at[pl.ds(wid * BPW, BPW)])


def _sc_mesh():
    return plsc.VectorSubcoreMesh(
        core_axis_name="c", subcore_axis_name="s",
        num_cores=NC, num_subcores=NS)


def _sc_gather_call(table, idx2d):
    _, d = table.shape

    def body(table_hbm, idx_hbm, out_hbm, idx_v, rows_v, gsem):
        wid = lax.axis_index("s") * NC + lax.axis_index("c")
        pltpu.sync_copy(idx_hbm.at[pl.ds(wid * BPW, BPW)], idx_v)
        _gather_waves(wid, table_hbm, idx_v, rows_v, out_hbm, gsem)

    fn = pl.kernel(
        body,
        out_type=_SDS((BPAD, d), jnp.float32),
        mesh=_sc_mesh(),
        scratch_types=[
            pltpu.VMEM((BPW,), jnp.int32),
            pltpu.VMEM((BPW, d), jnp.float32),
            pltpu.SemaphoreType.DMA,
        ],
        compiler_params=pltpu.CompilerParams(use_tc_tiling_on_sc=False),
    )
    return fn(table, idx2d.reshape(BPAD))


def _dot(a, b):
    return jnp.dot(a, b, preferred_element_type=jnp.float32)


def _selu(x):
    alpha = 1.6732631921768188
    scale = 1.0507010221481323
    return scale * jnp.where(x > 0, x, alpha * (jnp.exp(x) - 1.0))


def _gru_p(x, h, wz, wr, wh, rz, rr, rh, bz, br, bh):
    z = jax.nn.sigmoid(_dot(x, wz) + _dot(h, rz) + bz)
    r = jax.nn.sigmoid(_dot(x, wr) + _dot(h, rr) + br)
    hh = jnp.tanh(_dot(x, wh) + _dot(r * h, rh) + bh)
    return z * h + (1.0 - z) * hh


def _path_embed_body(x_ref, w1_ref, b1_ref, w2_ref, b2_ref, o_ref):
    h = _selu(_dot(x_ref[...], w1_ref[...]) + b1_ref[...])
    o_ref[...] = _selu(_dot(h, w2_ref[...]) + b2_ref[...])


def _link_embed_body(pgt_ref, cap_ref, mll_ref, deg_ref, msk_ref,
                     w1_ref, b1_ref, w2_ref, b2_ref, o_ref):
    acc = jnp.zeros((RL, 128), jnp.float32)
    for j in range(PPL):
        acc = acc + pgt_ref[pl.ds(j * RLS, RL), :]
    cap = cap_ref[...]
    load = acc / (cap * 1e9)
    nl = load / mll_ref[0, 0]
    lfeat = (cap * msk_ref[0:1, :] + load * msk_ref[1:2, :]
             + nl * msk_ref[2:3, :] + deg_ref[...] * msk_ref[3:4, :])
    h = _selu(_dot(lfeat, w1_ref[...]) + b1_ref[...])
    o_ref[...] = _selu(_dot(h, w2_ref[...]) + b2_ref[...])


def _scan_body(x_ref, h0_ref, wz_ref, wr_ref, wh_ref, rz_ref, rr_ref, rh_ref,
               bz_ref, br_ref, bh_ref, pss_ref, hout_ref):
    wz, wr, wh = wz_ref[...], wr_ref[...], wh_ref[...]
    rz, rr, rh = rz_ref[...], rr_ref[...], rh_ref[...]
    bz, br, bh = bz_ref[...], br_ref[...], bh_ref[...]
    h = h0_ref[...]
    pss_ref[pl.ds(0, RP), :] = h
    for t in range(T):
        x = x_ref[pl.ds(t * RPS, RP), :]
        h = _gru_p(x, h, wz, wr, wh, rz, rr, rh, bz, br, bh)
        pss_ref[pl.ds((t + 1) * RP, RP), :] = h
    hout_ref[...] = h


def _att_body(pg_ref, ls_ref, aw_ref, ab_ref, g_ref,
              wz_ref, wr_ref, wh_ref, rz_ref, rr_ref, rh_ref,
              bz_ref, br_ref, bh_ref, o_ref):
    aw = aw_ref[...]
    ab = ab_ref[...]
    gm = g_ref[...]
    m = jnp.zeros((RL, 128), jnp.float32)
    for c in range(8):
        blk = pg_ref[pl.ds(c * 10 * RLS, 10 * RLS), :]
        a = _dot(blk, aw) + ab
        a = jnp.where(a > 0, a, 0.01 * a)
        e = jnp.exp(a)
        w = (e / _dot(e, gm)) * blk
        for q in range(10):
            m = m + w[q * RLS:q * RLS + RL, :]
    o_ref[...] = _gru_p(m, ls_ref[...],
                        wz_ref[...], wr_ref[...], wh_ref[...],
                        rz_ref[...], rr_ref[...], rh_ref[...],
                        bz_ref[...], br_ref[...], bh_ref[...])


def _readout_body(pss_ref, capg_ref, k1w_ref, k1b_ref, k1s_ref,
                  k2w_ref, k2b_ref, k2s_ref, p_ref, o_ref):
    k1w, k1b = k1w_ref[...], k1b_ref[...]
    k2w, k2b = k2w_ref[...], k2b_ref[...]
    psel = p_ref[...]
    out = jnp.zeros((RP, 8), jnp.float32)
    for t in range(T):
        x = pss_ref[pl.ds((t + 1) * RP, RP), :]
        acc = _dot(jax.nn.silu(x), k1w) + k1b
        for g in range(8):
            phi = jnp.exp(-(x - GRIDPTS[g]) ** 2)
            acc = acc + _dot(phi, k1s_ref[g])
        occ = _dot(jax.nn.silu(acc), k2w) + k2b
        for g in range(8):
            phi = jnp.exp(-(acc - GRIDPTS[g]) ** 2)
            occ = occ + _dot(phi, k2s_ref[g])
        cap8 = _dot(capg_ref[pl.ds(t * RPS, RP), :], psel)
        out = out + occ / cap8
    o_ref[...] = out


def _tc(body, out_shape, *args):
    if isinstance(out_shape, list):
        shapes = [_SDS(s, jnp.float32) for s in out_shape]
    else:
        shapes = _SDS(out_shape, jnp.float32)
    return pl.pallas_call(body, out_shape=shapes)(*args)


def kernel(flow_traffic, flow_packets, flow_packet_size, flow_ibg,
           flow_on_rate, flow_p90PktSize, flow_bitrate_per_burst,
           flow_pkts_per_burst, link_capacity, max_link_load, flow_length,
           link_to_path, path_to_link, devices_to_link, pe_W1, pe_b1, pe_W2,
           pe_b2, le_W1, le_b1, le_W2, le_b2, att_W, att_b, pu_Wk, pu_Rk,
           pu_b, lu_Wk, lu_Rk, lu_b, k1_Wb, k1_Ws, k1_b, k2_Wb, k2_Ws, k2_b):
    f32 = jnp.float32
    i8 = jnp.eye(8, dtype=f32)
    bd = lambda w: jnp.kron(i8, w)
    t16 = lambda b: jnp.tile(b, 8)[None]

    l2p = link_to_path.astype(jnp.int32)
    idx_a = jnp.pad(l2p.T, ((0, 0), (0, PADP - NP))).reshape(NW * NCH, CH)
    f_idx = path_to_link[..., 0].astype(jnp.int32)
    t_idx = path_to_link[..., 1].astype(jnp.int32)
    idx_b = jnp.pad((t_idx * NP + f_idx).T,
                    ((0, 0), (0, PADL - NL))).reshape(NW * NCH, CH)
    idx_c = jnp.pad(f_idx.T, ((0, 0), (0, PADL - NL))).reshape(NW * NCH, CH)

    pfeat = jnp.concatenate(
        [flow_traffic, flow_packets, flow_packet_size, flow_ibg,
         flow_on_rate, flow_p90PktSize, flow_bitrate_per_burst,
         flow_pkts_per_burst, flow_length.astype(f32)[:, None]], axis=1)
    pfeat = jnp.pad(pfeat, ((0, 0), (0, 7))).reshape(RP, 128)
    ndeg = jnp.sum(jnp.ones_like(devices_to_link), axis=1)
    l2d = devices_to_link * 0 + jnp.arange(devices_to_link.shape[0])[:, None]
    gdeg = jnp.take(ndeg, l2d)
    degs = (jnp.reshape(gdeg, (-1,)) / jnp.sum(gdeg)).astype(f32)[:, None]

    ftb = jnp.broadcast_to(flow_traffic, (NP, H))
    capb = jnp.broadcast_to(link_capacity, (NL, H))
    capp = capb.reshape(RL, 128)
    degp = jnp.broadcast_to(degs, (NL, H)).reshape(RL, 128)
    msk = jnp.stack([jnp.tile((jnp.arange(16) == i).astype(f32), 8)
                     for i in range(4)])

    pe_w1 = bd(jnp.pad(pe_W1, ((0, 7), (0, 0))))
    pe_w2 = bd(pe_W2)
    le_w1 = bd(jnp.pad(le_W1, ((0, 12), (0, 0))))
    le_w2 = bd(le_W2)
    pu = [bd(pu_Wk[:, g * H:(g + 1) * H]) for g in range(3)] + \
         [bd(pu_Rk[:, g * H:(g + 1) * H]) for g in range(3)] + \
         [t16(pu_b[g * H:(g + 1) * H]) for g in range(3)]
    lu = [bd(lu_Wk[:, g * H:(g + 1) * H]) for g in range(3)] + \
         [bd(lu_Rk[:, g * H:(g + 1) * H]) for g in range(3)] + \
         [t16(lu_b[g * H:(g + 1) * H]) for g in range(3)]
    aw = bd(att_W)
    ab = t16(att_b)
    gmat = jnp.kron(i8, jnp.ones((16, 16), f32))
    k1w = bd(k1_Wb)
    k1b = jnp.tile(k1_b, 8)[None]
    k1s = jnp.stack([bd(k1_Ws[:, g, :]) for g in range(8)])
    k2w = bd(k2_Wb)
    k2b = jnp.tile(k2_b, 8)[None]
    k2s = jnp.stack([bd(k2_Ws[:, g, :]) for g in range(8)])
    psel = jnp.kron(i8, (jnp.arange(16) == 0).astype(f32)[:, None])

    pgt = _sc_gather_call(ftb, idx_c).reshape(BPAD // 8, 128)
    capg = _sc_gather_call(capb, idx_a).reshape(BPAD // 8, 128)
    ps = _tc(_path_embed_body, (RP, 128),
             pfeat, pe_w1, t16(pe_b1), pe_w2, t16(pe_b2))
    ls = _tc(_link_embed_body, (RL, 128),
             pgt, capp, max_link_load.reshape(1, 1), degp, msk,
             le_w1, t16(le_b1), le_w2, t16(le_b2))

    pss = None
    for _ in range(12):
        xg = _sc_gather_call(ls.reshape(NL, H), idx_a).reshape(BPAD // 8, 128)
        pss, ps = _tc(_scan_body, [(9 * RP, 128), (RP, 128)],
                      xg, ps, *pu)
        pg = _sc_gather_call(pss.reshape(9 * NP, H),
                             idx_b).reshape(BPAD // 8, 128)
        ls = _tc(_att_body, (RL, 128), pg, ls, aw, ab, gmat, *lu)

    out = _tc(_readout_body, (RP, 8),
              pss, capg, k1w, k1b, k1s, k2w, k2b, k2s, psel)
    return out.reshape(NP, 1)


# speedup vs baseline: 1.4559x; 1.2252x over previous
import numpy as np
import jax
import jax.numpy as jnp
from jax import lax
from jax.experimental import pallas as pl
from jax.experimental.pallas import tpu as pltpu
from jax.experimental.pallas import tpu_sc as plsc

H = 16
NP = 20000
NL = 2000
T = 8
PPL = 80
PADP = 20480
PADL = 2048
BPAD = 163840
RP = NP // 8
RPS = PADP // 8
RL = NL // 8
RLS = PADL // 8
NC, NS = 2, 16
NW = NC * NS
BPW = BPAD // NW
CH = 128
NCH = BPW // CH
GRIDPTS = np.linspace(-2.0, 2.0, 8).astype(np.float32)

_SDS = jax.ShapeDtypeStruct


def _gather_waves(wid, table_hbm, idx_v, rows_v, out_hbm, gsem):
    pltpu.async_copy(table_hbm.at[idx_v], rows_v, gsem).wait()
    pltpu.sync_copy(rows_v, out_hbm.at[pl.ds(wid * BPW, BPW)])


def _sc_mesh():
    return plsc.VectorSubcoreMesh(
        core_axis_name="c", subcore_axis_name="s",
        num_cores=NC, num_subcores=NS)


def _sc_gather_call(table, idx2d):
    v, d = table.shape
    if v * d * 4 <= 1024 * 1024:
        return _sc_gather_spmem_call(table, idx2d)

    def body(table_hbm, idx_hbm, out_hbm, idx_v, rows_v, gsem):
        wid = lax.axis_index("s") * NC + lax.axis_index("c")
        pltpu.sync_copy(idx_hbm.at[pl.ds(wid * BPW, BPW)], idx_v)
        _gather_waves(wid, table_hbm, idx_v, rows_v, out_hbm, gsem)

    fn = pl.kernel(
        body,
        out_type=_SDS((BPAD, d), jnp.float32),
        mesh=_sc_mesh(),
        scratch_types=[
            pltpu.VMEM((BPW,), jnp.int32),
            pltpu.VMEM((BPW, d), jnp.float32),
            pltpu.SemaphoreType.DMA,
        ],
        compiler_params=pltpu.CompilerParams(use_tc_tiling_on_sc=False),
    )
    return fn(table, idx2d.reshape(BPAD))


def _sc_gather_spmem_call(table, idx2d):
    v, d = table.shape

    def body(table_hbm, idx_hbm, out_hbm, tab_s, idx_v, rows_v, gsem):
        sid = lax.axis_index("s")
        wid = sid * NC + lax.axis_index("c")

        @pl.when(sid == 0)
        def _stage():
            pltpu.sync_copy(table_hbm, tab_s)

        pltpu.sync_copy(idx_hbm.at[pl.ds(wid * BPW, BPW)], idx_v)
        plsc.subcore_barrier()
        pltpu.async_copy(tab_s.at[idx_v], rows_v, gsem).wait()
        pltpu.sync_copy(rows_v, out_hbm.at[pl.ds(wid * BPW, BPW)])

    fn = pl.kernel(
        body,
        out_type=_SDS((BPAD, d), jnp.float32),
        mesh=_sc_mesh(),
        scratch_types=[
            pltpu.VMEM_SHARED((v, d), jnp.float32),
            pltpu.VMEM((BPW,), jnp.int32),
            pltpu.VMEM((BPW, d), jnp.float32),
            pltpu.SemaphoreType.DMA,
        ],
        compiler_params=pltpu.CompilerParams(use_tc_tiling_on_sc=False),
    )
    return fn(table, idx2d.reshape(BPAD))


def _dot(a, b):
    return jnp.dot(a, b, preferred_element_type=jnp.float32)


def _selu(x):
    alpha = 1.6732631921768188
    scale = 1.0507010221481323
    return scale * jnp.where(x > 0, x, alpha * (jnp.exp(x) - 1.0))


def _gru_p(x, h, wz, wr, wh, rz, rr, rh, bz, br, bh):
    z = jax.nn.sigmoid(_dot(x, wz) + _dot(h, rz) + bz)
    r = jax.nn.sigmoid(_dot(x, wr) + _dot(h, rr) + br)
    hh = jnp.tanh(_dot(x, wh) + _dot(r * h, rh) + bh)
    return z * h + (1.0 - z) * hh


def _path_embed_body(x_ref, w1_ref, b1_ref, w2_ref, b2_ref, o_ref):
    h = _selu(_dot(x_ref[...], w1_ref[...]) + b1_ref[...])
    o_ref[...] = _selu(_dot(h, w2_ref[...]) + b2_ref[...])


def _link_embed_body(pgt_ref, cap_ref, mll_ref, deg_ref, msk_ref,
                     w1_ref, b1_ref, w2_ref, b2_ref, o_ref):
    acc = jnp.zeros((RL, 128), jnp.float32)
    for j in range(PPL):
        acc = acc + pgt_ref[pl.ds(j * RLS, RL), :]
    cap = cap_ref[...]
    load = acc / (cap * 1e9)
    nl = load / mll_ref[0, 0]
    lfeat = (cap * msk_ref[0:1, :] + load * msk_ref[1:2, :]
             + nl * msk_ref[2:3, :] + deg_ref[...] * msk_ref[3:4, :])
    h = _selu(_dot(lfeat, w1_ref[...]) + b1_ref[...])
    o_ref[...] = _selu(_dot(h, w2_ref[...]) + b2_ref[...])


def _scan_body(x_ref, h0_ref, wz_ref, wr_ref, wh_ref, rz_ref, rr_ref, rh_ref,
               bz_ref, br_ref, bh_ref, pss_ref, hout_ref):
    wz, wr, wh = wz_ref[...], wr_ref[...], wh_ref[...]
    rz, rr, rh = rz_ref[...], rr_ref[...], rh_ref[...]
    bz, br, bh = bz_ref[...], br_ref[...], bh_ref[...]
    h = h0_ref[...]
    pss_ref[pl.ds(0, RP), :] = h
    for t in range(T):
        x = x_ref[pl.ds(t * RPS, RP), :]
        h = _gru_p(x, h, wz, wr, wh, rz, rr, rh, bz, br, bh)
        pss_ref[pl.ds((t + 1) * RP, RP), :] = h
    hout_ref[...] = h


def _att_body(pg_ref, ls_ref, aw_ref, ab_ref, g_ref,
              wz_ref, wr_ref, wh_ref, rz_ref, rr_ref, rh_ref,
              bz_ref, br_ref, bh_ref, o_ref):
    aw = aw_ref[...]
    ab = ab_ref[...]
    gm = g_ref[...]
    m = jnp.zeros((RL, 128), jnp.float32)
    for c in range(8):
        blk = pg_ref[pl.ds(c * 10 * RLS, 10 * RLS), :]
        a = _dot(blk, aw) + ab
        a = jnp.where(a > 0, a, 0.01 * a)
        e = jnp.exp(a)
        w = (e / _dot(e, gm)) * blk
        for q in range(10):
            m = m + w[q * RLS:q * RLS + RL, :]
    o_ref[...] = _gru_p(m, ls_ref[...],
                        wz_ref[...], wr_ref[...], wh_ref[...],
                        rz_ref[...], rr_ref[...], rh_ref[...],
                        bz_ref[...], br_ref[...], bh_ref[...])


def _readout_body(pss_ref, capg_ref, k1w_ref, k1b_ref, k1s_ref,
                  k2w_ref, k2b_ref, k2s_ref, p_ref, o_ref):
    k1w, k1b = k1w_ref[...], k1b_ref[...]
    k2w, k2b = k2w_ref[...], k2b_ref[...]
    psel = p_ref[...]
    out = jnp.zeros((RP, 8), jnp.float32)
    for t in range(T):
        x = pss_ref[pl.ds((t + 1) * RP, RP), :]
        acc = _dot(jax.nn.silu(x), k1w) + k1b
        for g in range(8):
            phi = jnp.exp(-(x - GRIDPTS[g]) ** 2)
            acc = acc + _dot(phi, k1s_ref[g])
        occ = _dot(jax.nn.silu(acc), k2w) + k2b
        for g in range(8):
            phi = jnp.exp(-(acc - GRIDPTS[g]) ** 2)
            occ = occ + _dot(phi, k2s_ref[g])
        cap8 = _dot(capg_ref[pl.ds(t * RPS, RP), :], psel)
        out = out + occ / cap8
    o_ref[...] = out


def _tc(body, out_shape, *args):
    if isinstance(out_shape, list):
        shapes = [_SDS(s, jnp.float32) for s in out_shape]
    else:
        shapes = _SDS(out_shape, jnp.float32)
    return pl.pallas_call(body, out_shape=shapes)(*args)


def kernel(flow_traffic, flow_packets, flow_packet_size, flow_ibg,
           flow_on_rate, flow_p90PktSize, flow_bitrate_per_burst,
           flow_pkts_per_burst, link_capacity, max_link_load, flow_length,
           link_to_path, path_to_link, devices_to_link, pe_W1, pe_b1, pe_W2,
           pe_b2, le_W1, le_b1, le_W2, le_b2, att_W, att_b, pu_Wk, pu_Rk,
           pu_b, lu_Wk, lu_Rk, lu_b, k1_Wb, k1_Ws, k1_b, k2_Wb, k2_Ws, k2_b):
    f32 = jnp.float32
    i8 = jnp.eye(8, dtype=f32)
    bd = lambda w: jnp.kron(i8, w)
    t16 = lambda b: jnp.tile(b, 8)[None]

    l2p = link_to_path.astype(jnp.int32)
    idx_a = jnp.pad(l2p.T, ((0, 0), (0, PADP - NP))).reshape(NW * NCH, CH)
    f_idx = path_to_link[..., 0].astype(jnp.int32)
    t_idx = path_to_link[..., 1].astype(jnp.int32)
    idx_b = jnp.pad((t_idx * NP + f_idx).T,
                    ((0, 0), (0, PADL - NL))).reshape(NW * NCH, CH)
    idx_c = jnp.pad(f_idx.T, ((0, 0), (0, PADL - NL))).reshape(NW * NCH, CH)

    pfeat = jnp.concatenate(
        [flow_traffic, flow_packets, flow_packet_size, flow_ibg,
         flow_on_rate, flow_p90PktSize, flow_bitrate_per_burst,
         flow_pkts_per_burst, flow_length.astype(f32)[:, None]], axis=1)
    pfeat = jnp.pad(pfeat, ((0, 0), (0, 7))).reshape(RP, 128)
    ndeg = jnp.sum(jnp.ones_like(devices_to_link), axis=1)
    l2d = devices_to_link * 0 + jnp.arange(devices_to_link.shape[0])[:, None]
    gdeg = jnp.take(ndeg, l2d)
    degs = (jnp.reshape(gdeg, (-1,)) / jnp.sum(gdeg)).astype(f32)[:, None]

    ftb = jnp.broadcast_to(flow_traffic, (NP, H))
    capb = jnp.broadcast_to(link_capacity, (NL, H))
    capp = capb.reshape(RL, 128)
    degp = jnp.broadcast_to(degs, (NL, H)).reshape(RL, 128)
    msk = jnp.stack([jnp.tile((jnp.arange(16) == i).astype(f32), 8)
                     for i in range(4)])

    pe_w1 = bd(jnp.pad(pe_W1, ((0, 7), (0, 0))))
    pe_w2 = bd(pe_W2)
    le_w1 = bd(jnp.pad(le_W1, ((0, 12), (0, 0))))
    le_w2 = bd(le_W2)
    pu = [bd(pu_Wk[:, g * H:(g + 1) * H]) for g in range(3)] + \
         [bd(pu_Rk[:, g * H:(g + 1) * H]) for g in range(3)] + \
         [t16(pu_b[g * H:(g + 1) * H]) for g in range(3)]
    lu = [bd(lu_Wk[:, g * H:(g + 1) * H]) for g in range(3)] + \
         [bd(lu_Rk[:, g * H:(g + 1) * H]) for g in range(3)] + \
         [t16(lu_b[g * H:(g + 1) * H]) for g in range(3)]
    aw = bd(att_W)
    ab = t16(att_b)
    gmat = jnp.kron(i8, jnp.ones((16, 16), f32))
    k1w = bd(k1_Wb)
    k1b = jnp.tile(k1_b, 8)[None]
    k1s = jnp.stack([bd(k1_Ws[:, g, :]) for g in range(8)])
    k2w = bd(k2_Wb)
    k2b = jnp.tile(k2_b, 8)[None]
    k2s = jnp.stack([bd(k2_Ws[:, g, :]) for g in range(8)])
    psel = jnp.kron(i8, (jnp.arange(16) == 0).astype(f32)[:, None])

    pgt = _sc_gather_call(ftb, idx_c).reshape(BPAD // 8, 128)
    capg = _sc_gather_call(capb, idx_a).reshape(BPAD // 8, 128)
    ps = _tc(_path_embed_body, (RP, 128),
             pfeat, pe_w1, t16(pe_b1), pe_w2, t16(pe_b2))
    ls = _tc(_link_embed_body, (RL, 128),
             pgt, capp, max_link_load.reshape(1, 1), degp, msk,
             le_w1, t16(le_b1), le_w2, t16(le_b2))

    pss = None
    for _ in range(12):
        xg = _sc_gather_call(ls.reshape(NL, H), idx_a).reshape(BPAD // 8, 128)
        pss, ps = _tc(_scan_body, [(9 * RP, 128), (RP, 128)],
                      xg, ps, *pu)
        pg = _sc_gather_call(pss.reshape(9 * NP, H),
                             idx_b).reshape(BPAD // 8, 128)
        ls = _tc(_att_body, (RL, 128), pg, ls, aw, ab, gmat, *lu)

    out = _tc(_readout_body, (RP, 8),
              pss, capg, k1w, k1b, k1s, k2w, k2b, k2s, psel)
    return out.reshape(NP, 1)


# speedup vs baseline: 1.4567x; 1.0006x over previous
import numpy as np
import jax
import jax.numpy as jnp
from jax import lax
from jax.experimental import pallas as pl
from jax.experimental.pallas import tpu as pltpu
from jax.experimental.pallas import tpu_sc as plsc

H = 16
NP = 20000
NL = 2000
T = 8
PPL = 80
PADP = 20480
PADL = 2048
BPAD = 163840
RP = NP // 8
RPS = PADP // 8
RL = NL // 8
RLS = PADL // 8
NC, NS = 2, 16
NW = NC * NS
BPW = BPAD // NW
CH = 128
NCH = BPW // CH
GRIDPTS = np.linspace(-2.0, 2.0, 8).astype(np.float32)

_SDS = jax.ShapeDtypeStruct


def _gather_waves(wid, table_hbm, idx_v, rows_v, out_hbm, gsem):
    pltpu.async_copy(table_hbm.at[idx_v], rows_v, gsem).wait()
    pltpu.sync_copy(rows_v, out_hbm.at[pl.ds(wid * BPW, BPW)])


def _sc_mesh():
    return plsc.VectorSubcoreMesh(
        core_axis_name="c", subcore_axis_name="s",
        num_cores=NC, num_subcores=NS)


def _sc_gather_call(table, idx2d):
    v, d = table.shape
    if v * d * 4 <= 4 * 1024 * 1024:
        return _sc_gather_spmem_call(table, idx2d)

    def body(table_hbm, idx_hbm, out_hbm, idx_v, rows_v, gsem):
        wid = lax.axis_index("s") * NC + lax.axis_index("c")
        pltpu.sync_copy(idx_hbm.at[pl.ds(wid * BPW, BPW)], idx_v)
        _gather_waves(wid, table_hbm, idx_v, rows_v, out_hbm, gsem)

    fn = pl.kernel(
        body,
        out_type=_SDS((BPAD, d), jnp.float32),
        mesh=_sc_mesh(),
        scratch_types=[
            pltpu.VMEM((BPW,), jnp.int32),
            pltpu.VMEM((BPW, d), jnp.float32),
            pltpu.SemaphoreType.DMA,
        ],
        compiler_params=pltpu.CompilerParams(use_tc_tiling_on_sc=False),
    )
    return fn(table, idx2d.reshape(BPAD))


def _sc_gather_spmem_call(table, idx2d):
    v, d = table.shape

    def body(table_hbm, idx_hbm, out_hbm, tab_s, idx_v, rows_v, gsem):
        sid = lax.axis_index("s")
        wid = sid * NC + lax.axis_index("c")

        @pl.when(sid == 0)
        def _stage():
            pltpu.sync_copy(table_hbm, tab_s)

        pltpu.sync_copy(idx_hbm.at[pl.ds(wid * BPW, BPW)], idx_v)
        plsc.subcore_barrier()
        pltpu.async_copy(tab_s.at[idx_v], rows_v, gsem).wait()
        pltpu.sync_copy(rows_v, out_hbm.at[pl.ds(wid * BPW, BPW)])

    fn = pl.kernel(
        body,
        out_type=_SDS((BPAD, d), jnp.float32),
        mesh=_sc_mesh(),
        scratch_types=[
            pltpu.VMEM_SHARED((v, d), jnp.float32),
            pltpu.VMEM((BPW,), jnp.int32),
            pltpu.VMEM((BPW, d), jnp.float32),
            pltpu.SemaphoreType.DMA,
        ],
        compiler_params=pltpu.CompilerParams(use_tc_tiling_on_sc=False),
    )
    return fn(table, idx2d.reshape(BPAD))


def _dot(a, b):
    return jnp.dot(a, b, preferred_element_type=jnp.float32)


def _selu(x):
    alpha = 1.6732631921768188
    scale = 1.0507010221481323
    return scale * jnp.where(x > 0, x, alpha * (jnp.exp(x) - 1.0))


def _gru_p(x, h, wz, wr, wh, rz, rr, rh, bz, br, bh):
    z = jax.nn.sigmoid(_dot(x, wz) + _dot(h, rz) + bz)
    r = jax.nn.sigmoid(_dot(x, wr) + _dot(h, rr) + br)
    hh = jnp.tanh(_dot(x, wh) + _dot(r * h, rh) + bh)
    return z * h + (1.0 - z) * hh


def _path_embed_body(x_ref, w1_ref, b1_ref, w2_ref, b2_ref, o_ref):
    h = _selu(_dot(x_ref[...], w1_ref[...]) + b1_ref[...])
    o_ref[...] = _selu(_dot(h, w2_ref[...]) + b2_ref[...])


def _link_embed_body(pgt_ref, cap_ref, mll_ref, deg_ref, msk_ref,
                     w1_ref, b1_ref, w2_ref, b2_ref, o_ref):
    acc = jnp.zeros((RL, 128), jnp.float32)
    for j in range(PPL):
        acc = acc + pgt_ref[pl.ds(j * RLS, RL), :]
    cap = cap_ref[...]
    load = acc / (cap * 1e9)
    nl = load / mll_ref[0, 0]
    lfeat = (cap * msk_ref[0:1, :] + load * msk_ref[1:2, :]
             + nl * msk_ref[2:3, :] + deg_ref[...] * msk_ref[3:4, :])
    h = _selu(_dot(lfeat, w1_ref[...]) + b1_ref[...])
    o_ref[...] = _selu(_dot(h, w2_ref[...]) + b2_ref[...])


def _scan_body(x_ref, h0_ref, wz_ref, wr_ref, wh_ref, rz_ref, rr_ref, rh_ref,
               bz_ref, br_ref, bh_ref, pss_ref, hout_ref):
    wz, wr, wh = wz_ref[...], wr_ref[...], wh_ref[...]
    rz, rr, rh = rz_ref[...], rr_ref[...], rh_ref[...]
    bz, br, bh = bz_ref[...], br_ref[...], bh_ref[...]
    h = h0_ref[...]
    pss_ref[pl.ds(0, RP), :] = h
    for t in range(T):
        x = x_ref[pl.ds(t * RPS, RP), :]
        h = _gru_p(x, h, wz, wr, wh, rz, rr, rh, bz, br, bh)
        pss_ref[pl.ds((t + 1) * RP, RP), :] = h
    hout_ref[...] = h


def _att_body(pg_ref, ls_ref, aw_ref, ab_ref, g_ref,
              wz_ref, wr_ref, wh_ref, rz_ref, rr_ref, rh_ref,
              bz_ref, br_ref, bh_ref, o_ref):
    aw = aw_ref[...]
    ab = ab_ref[...]
    gm = g_ref[...]
    m = jnp.zeros((RL, 128), jnp.float32)
    for c in range(8):
        blk = pg_ref[pl.ds(c * 10 * RLS, 10 * RLS), :]
        a = _dot(blk, aw) + ab
        a = jnp.where(a > 0, a, 0.01 * a)
        e = jnp.exp(a)
        w = (e / _dot(e, gm)) * blk
        for q in range(10):
            m = m + w[q * RLS:q * RLS + RL, :]
    o_ref[...] = _gru_p(m, ls_ref[...],
                        wz_ref[...], wr_ref[...], wh_ref[...],
                        rz_ref[...], rr_ref[...], rh_ref[...],
                        bz_ref[...], br_ref[...], bh_ref[...])


def _readout_body(pss_ref, capg_ref, k1w_ref, k1b_ref, k1s_ref,
                  k2w_ref, k2b_ref, k2s_ref, p_ref, o_ref):
    k1w, k1b = k1w_ref[...], k1b_ref[...]
    k2w, k2b = k2w_ref[...], k2b_ref[...]
    psel = p_ref[...]
    out = jnp.zeros((RP, 8), jnp.float32)
    for t in range(T):
        x = pss_ref[pl.ds((t + 1) * RP, RP), :]
        acc = _dot(jax.nn.silu(x), k1w) + k1b
        for g in range(8):
            phi = jnp.exp(-(x - GRIDPTS[g]) ** 2)
            acc = acc + _dot(phi, k1s_ref[g])
        occ = _dot(jax.nn.silu(acc), k2w) + k2b
        for g in range(8):
            phi = jnp.exp(-(acc - GRIDPTS[g]) ** 2)
            occ = occ + _dot(phi, k2s_ref[g])
        cap8 = _dot(capg_ref[pl.ds(t * RPS, RP), :], psel)
        out = out + occ / cap8
    o_ref[...] = out


def _tc(body, out_shape, *args):
    if isinstance(out_shape, list):
        shapes = [_SDS(s, jnp.float32) for s in out_shape]
    else:
        shapes = _SDS(out_shape, jnp.float32)
    return pl.pallas_call(body, out_shape=shapes)(*args)


def kernel(flow_traffic, flow_packets, flow_packet_size, flow_ibg,
           flow_on_rate, flow_p90PktSize, flow_bitrate_per_burst,
           flow_pkts_per_burst, link_capacity, max_link_load, flow_length,
           link_to_path, path_to_link, devices_to_link, pe_W1, pe_b1, pe_W2,
           pe_b2, le_W1, le_b1, le_W2, le_b2, att_W, att_b, pu_Wk, pu_Rk,
           pu_b, lu_Wk, lu_Rk, lu_b, k1_Wb, k1_Ws, k1_b, k2_Wb, k2_Ws, k2_b):
    f32 = jnp.float32
    i8 = jnp.eye(8, dtype=f32)
    bd = lambda w: jnp.kron(i8, w)
    t16 = lambda b: jnp.tile(b, 8)[None]

    l2p = link_to_path.astype(jnp.int32)
    idx_a = jnp.pad(l2p.T, ((0, 0), (0, PADP - NP))).reshape(NW * NCH, CH)
    f_idx = path_to_link[..., 0].astype(jnp.int32)
    t_idx = path_to_link[..., 1].astype(jnp.int32)
    idx_b = jnp.pad((t_idx * NP + f_idx).T,
                    ((0, 0), (0, PADL - NL))).reshape(NW * NCH, CH)
    idx_c = jnp.pad(f_idx.T, ((0, 0), (0, PADL - NL))).reshape(NW * NCH, CH)

    pfeat = jnp.concatenate(
        [flow_traffic, flow_packets, flow_packet_size, flow_ibg,
         flow_on_rate, flow_p90PktSize, flow_bitrate_per_burst,
         flow_pkts_per_burst, flow_length.astype(f32)[:, None]], axis=1)
    pfeat = jnp.pad(pfeat, ((0, 0), (0, 7))).reshape(RP, 128)
    ndeg = jnp.sum(jnp.ones_like(devices_to_link), axis=1)
    l2d = devices_to_link * 0 + jnp.arange(devices_to_link.shape[0])[:, None]
    gdeg = jnp.take(ndeg, l2d)
    degs = (jnp.reshape(gdeg, (-1,)) / jnp.sum(gdeg)).astype(f32)[:, None]

    ftb = jnp.broadcast_to(flow_traffic, (NP, H))
    capb = jnp.broadcast_to(link_capacity, (NL, H))
    capp = capb.reshape(RL, 128)
    degp = jnp.broadcast_to(degs, (NL, H)).reshape(RL, 128)
    msk = jnp.stack([jnp.tile((jnp.arange(16) == i).astype(f32), 8)
                     for i in range(4)])

    pe_w1 = bd(jnp.pad(pe_W1, ((0, 7), (0, 0))))
    pe_w2 = bd(pe_W2)
    le_w1 = bd(jnp.pad(le_W1, ((0, 12), (0, 0))))
    le_w2 = bd(le_W2)
    pu = [bd(pu_Wk[:, g * H:(g + 1) * H]) for g in range(3)] + \
         [bd(pu_Rk[:, g * H:(g + 1) * H]) for g in range(3)] + \
         [t16(pu_b[g * H:(g + 1) * H]) for g in range(3)]
    lu = [bd(lu_Wk[:, g * H:(g + 1) * H]) for g in range(3)] + \
         [bd(lu_Rk[:, g * H:(g + 1) * H]) for g in range(3)] + \
         [t16(lu_b[g * H:(g + 1) * H]) for g in range(3)]
    aw = bd(att_W)
    ab = t16(att_b)
    gmat = jnp.kron(i8, jnp.ones((16, 16), f32))
    k1w = bd(k1_Wb)
    k1b = jnp.tile(k1_b, 8)[None]
    k1s = jnp.stack([bd(k1_Ws[:, g, :]) for g in range(8)])
    k2w = bd(k2_Wb)
    k2b = jnp.tile(k2_b, 8)[None]
    k2s = jnp.stack([bd(k2_Ws[:, g, :]) for g in range(8)])
    psel = jnp.kron(i8, (jnp.arange(16) == 0).astype(f32)[:, None])

    pgt = _sc_gather_call(ftb, idx_c).reshape(BPAD // 8, 128)
    capg = _sc_gather_call(capb, idx_a).reshape(BPAD // 8, 128)
    ps = _tc(_path_embed_body, (RP, 128),
             pfeat, pe_w1, t16(pe_b1), pe_w2, t16(pe_b2))
    ls = _tc(_link_embed_body, (RL, 128),
             pgt, capp, max_link_load.reshape(1, 1), degp, msk,
             le_w1, t16(le_b1), le_w2, t16(le_b2))

    pss = None
    for _ in range(12):
        xg = _sc_gather_call(ls.reshape(NL, H), idx_a).reshape(BPAD // 8, 128)
        pss, ps = _tc(_scan_body, [(9 * RP, 128), (RP, 128)],
                      xg, ps, *pu)
        pg = _sc_gather_call(pss.reshape(9 * NP, H),
                             idx_b).reshape(BPAD // 8, 128)
        ls = _tc(_att_body, (RL, 128), pg, ls, aw, ab, gmat, *lu)

    out = _tc(_readout_body, (RP, 8),
              pss, capg, k1w, k1b, k1s, k2w, k2b, k2s, psel)
    return out.reshape(NP, 1)
